# baseline (speedup 1.0000x reference)
import jax
import jax.numpy as jnp
from jax.experimental import pallas as pl

_BLOCK_ROWS = 1024


def _bias_broadcast_kernel(b_ref, o_ref):
    o_ref[...] = jnp.broadcast_to(b_ref[...], o_ref.shape)


def kernel(x, W, b):
    batch = x.shape[0]
    out_dim = b.shape[0]
    block_rows = min(_BLOCK_ROWS, batch)
    b2 = b.reshape(1, out_dim)
    return pl.pallas_call(
        _bias_broadcast_kernel,
        grid=(batch // block_rows,),
        in_specs=[pl.BlockSpec((1, out_dim), lambda i: (0, 0))],
        out_specs=pl.BlockSpec((block_rows, out_dim), lambda i: (i, 0)),
        out_shape=jax.ShapeDtypeStruct((batch, out_dim), x.dtype),
    )(b2)


# speedup vs baseline: 1.0709x; 1.0709x over previous
import jax
import jax.numpy as jnp
from jax.experimental import pallas as pl
from jax.experimental.pallas import tpu as pltpu

_TILE_ROWS = 512


def _make_body(batch, out_dim, tile_rows):
    n_copies = batch // tile_rows

    def body(b_ref, o_ref, tile, sem):
        tile[...] = jnp.broadcast_to(b_ref[...], (tile_rows, out_dim))
        copies = [
            pltpu.make_async_copy(
                tile,
                o_ref.at[pl.ds(k * tile_rows, tile_rows), :],
                sem.at[k],
            )
            for k in range(n_copies)
        ]
        for c in copies:
            c.start()
        for c in copies:
            c.wait()

    return body, n_copies


def kernel(x, W, b):
    batch = x.shape[0]
    out_dim = b.shape[0]
    tile_rows = min(_TILE_ROWS, batch)
    body, n_copies = _make_body(batch, out_dim, tile_rows)
    b2 = b.reshape(1, out_dim)
    return pl.pallas_call(
        body,
        in_specs=[pl.BlockSpec(memory_space=pltpu.MemorySpace.VMEM)],
        out_specs=pl.BlockSpec(memory_space=pltpu.MemorySpace.HBM),
        out_shape=jax.ShapeDtypeStruct((batch, out_dim), x.dtype),
        scratch_shapes=[
            pltpu.VMEM((tile_rows, out_dim), jnp.float32),
            pltpu.SemaphoreType.DMA((n_copies,)),
        ],
    )(b2)


# speedup vs baseline: 1.0839x; 1.0121x over previous
import jax
import jax.numpy as jnp
from jax.experimental import pallas as pl
from jax.experimental.pallas import tpu as pltpu

_TILE_ROWS = 2048


def _make_body(batch, out_dim, tile_rows):
    n_copies = batch // tile_rows

    def body(b_ref, o_ref, tile, sem):
        tile[...] = jnp.broadcast_to(b_ref[...], (tile_rows, out_dim))
        copies = [
            pltpu.make_async_copy(
                tile,
                o_ref.at[pl.ds(k * tile_rows, tile_rows), :],
                sem.at[k],
            )
            for k in range(n_copies)
        ]
        for c in copies:
            c.start()
        for c in copies:
            c.wait()

    return body, n_copies


def kernel(x, W, b):
    batch = x.shape[0]
    out_dim = b.shape[0]
    tile_rows = min(_TILE_ROWS, batch)
    body, n_copies = _make_body(batch, out_dim, tile_rows)
    b2 = b.reshape(1, out_dim)
    return pl.pallas_call(
        body,
        in_specs=[pl.BlockSpec(memory_space=pltpu.MemorySpace.VMEM)],
        out_specs=pl.BlockSpec(memory_space=pltpu.MemorySpace.HBM),
        out_shape=jax.ShapeDtypeStruct((batch, out_dim), x.dtype),
        scratch_shapes=[
            pltpu.VMEM((tile_rows, out_dim), jnp.float32),
            pltpu.SemaphoreType.DMA((n_copies,)),
        ],
    )(b2)
